# baseline (speedup 1.0000x reference)
import functools

import jax
import jax.numpy as jnp
from jax import lax
from jax.experimental import pallas as pl
from jax.experimental.pallas import tpu as pltpu
from jax.experimental.pallas import tpu_sc as plsc

N_NODES = 10000
N_PAD = 10240
D_FEAT = 128
N_EDGES = 320000
K_HOPS = 3
ALPHA = 0.05

NC = 2
NS = 16
NW = NC * NS
E_PER_W = N_EDGES // NW
GK = 80
NG = E_PER_W // GK
HGK = 72
NSLAB = 4
SGRP = 36
NBUF = 4
E_PAD = NW * NSLAB * SGRP * HGK
ROWS_S = N_PAD // NS
ROWS_W = N_PAD // NW

_MESH = plsc.VectorSubcoreMesh(core_axis_name="c", subcore_axis_name="s")


def _wid():
    return lax.axis_index("s") * NC + lax.axis_index("c")


def _fill(ref, nrows, value):
    ncol = ref.shape[1] // 16
    v = jnp.full((16,), value, jnp.float32)

    def body(r, carry):
        for j in range(ncol):
            ref[r, pl.ds(j * 16, 16)] = v
        return carry

    lax.fori_loop(0, nrows, body, 0)


@functools.partial(
    pl.kernel,
    out_type=jax.ShapeDtypeStruct((NC, N_PAD, D_FEAT), jnp.float32),
    mesh=_MESH,
    scratch_types=[
        pltpu.VMEM((NG, GK), jnp.int32),
        pltpu.VMEM((GK, D_FEAT), jnp.float32),
        pltpu.VMEM_SHARED((N_PAD, D_FEAT), jnp.float32),
        pltpu.SemaphoreType.DMA,
    ],
)
def _deg_kernel(col3, degp, cidx, ones_v, hist, sem):
    cid = lax.axis_index("c")
    sid = lax.axis_index("s")
    w = _wid()
    pltpu.sync_copy(col3.at[w], cidx)
    _fill(ones_v, GK, 0.0)
    for q in range(ROWS_S // GK):
        pltpu.sync_copy(ones_v, hist.at[pl.ds(sid * ROWS_S + q * GK, GK)])
    plsc.subcore_barrier()
    _fill(ones_v, GK, 1.0)

    def body(g, carry):
        pltpu.sync_copy(ones_v, hist.at[cidx.at[g]], add=True)
        return carry

    lax.fori_loop(0, NG, body, 0)
    plsc.subcore_barrier()
    pltpu.sync_copy(hist.at[pl.ds(sid * ROWS_S, ROWS_S)],
                    degp.at[cid, pl.ds(sid * ROWS_S, ROWS_S)])


@functools.partial(
    pl.kernel,
    out_type=(
        jax.ShapeDtypeStruct((N_PAD, D_FEAT), jnp.float32),
        jax.ShapeDtypeStruct((N_PAD, D_FEAT), jnp.float32),
    ),
    mesh=_MESH,
    scratch_types=[
        pltpu.VMEM((GK, D_FEAT), jnp.float32),
        pltpu.VMEM((GK, D_FEAT), jnp.float32),
        pltpu.VMEM((GK, D_FEAT), jnp.float32),
        pltpu.SemaphoreType.DMA,
    ],
)
def _init_kernel(x, dis128, out, hhat, xb, db, hb, sem):
    w = _wid()

    for q in range(ROWS_W // GK):
        base = w * ROWS_W + q * GK
        pltpu.sync_copy(x.at[pl.ds(base, GK)], xb)
        pltpu.sync_copy(dis128.at[pl.ds(base, GK)], db)

        def body(r, carry):
            for j in range(D_FEAT // 16):
                s = pl.ds(j * 16, 16)
                v = xb[r, s]
                hb[r, s] = db[r, s] * v
                xb[r, s] = ALPHA * v
            return carry

        lax.fori_loop(0, GK, body, 0)
        pltpu.sync_copy(xb, out.at[pl.ds(base, GK)])
        pltpu.sync_copy(hb, hhat.at[pl.ds(base, GK)])


def _slab_phase(hhat, rs, cs, bufs, acc, gsems, ssems):
    def body(i, carry):
        g0 = i * NBUF
        for k in range(NBUF):
            pltpu.make_async_copy(hhat.at[pl.ds(0, HGK)], bufs[k],
                                  gsems[k]).wait()
            pltpu.async_copy(bufs[k], acc.at[cs.at[g0 + k]], ssems[k],
                             add=True)
        for k in range(NBUF):
            pltpu.make_async_copy(bufs[k], acc.at[pl.ds(0, HGK)],
                                  ssems[k]).wait()
            gn = jnp.minimum(g0 + NBUF + k, SGRP - 1)
            pltpu.async_copy(hhat.at[rs.at[gn]], bufs[k], gsems[k])
        return carry

    lax.fori_loop(0, SGRP // NBUF, body, 0)


@functools.partial(
    pl.kernel,
    out_type=jax.ShapeDtypeStruct((NC, N_PAD, D_FEAT), jnp.float32),
    mesh=_MESH,
    scratch_types=[
        pltpu.VMEM((SGRP, HGK), jnp.int32),
        pltpu.VMEM((SGRP, HGK), jnp.int32),
    ] + [pltpu.VMEM((HGK, D_FEAT), jnp.float32)] * NBUF
      + [pltpu.VMEM_SHARED((N_PAD, D_FEAT), jnp.float32)]
      + [pltpu.SemaphoreType.DMA] * (2 * NBUF),
)
def _hop_kernel(hhat, row4, col4, part, rs, cs, *rest):
    bufs = rest[:NBUF]
    acc = rest[NBUF]
    gsems = rest[NBUF + 1:NBUF + 1 + NBUF]
    ssems = rest[NBUF + 1 + NBUF:]
    cid = lax.axis_index("c")
    sid = lax.axis_index("s")
    w = _wid()
    _fill(bufs[0], HGK, 0.0)
    for q in range(ROWS_S // HGK):
        pltpu.sync_copy(bufs[0], acc.at[pl.ds(sid * ROWS_S + q * HGK, HGK)])
    rem = ROWS_S - (ROWS_S // HGK) * HGK
    if rem:
        pltpu.sync_copy(
            bufs[0].at[pl.ds(0, rem)],
            acc.at[pl.ds(sid * ROWS_S + (ROWS_S // HGK) * HGK, rem)])
    plsc.subcore_barrier()

    for s in range(NSLAB):
        pltpu.sync_copy(row4.at[w, s], rs)
        pltpu.sync_copy(col4.at[w, s], cs)
        for k in range(NBUF):
            pltpu.async_copy(hhat.at[rs.at[k]], bufs[k], gsems[k])
        _slab_phase(hhat, rs, cs, bufs, acc, gsems, ssems)
        for k in range(NBUF):
            pltpu.make_async_copy(hhat.at[pl.ds(0, HGK)], bufs[k],
                                  gsems[k]).wait()

    plsc.subcore_barrier()
    pltpu.sync_copy(acc.at[pl.ds(sid * ROWS_S, ROWS_S)],
                    part.at[cid, pl.ds(sid * ROWS_S, ROWS_S)])


@functools.partial(
    pl.kernel,
    out_type=(
        jax.ShapeDtypeStruct((N_PAD, D_FEAT), jnp.float32),
        jax.ShapeDtypeStruct((N_PAD, D_FEAT), jnp.float32),
    ),
    mesh=_MESH,
    scratch_types=[
        pltpu.VMEM((GK, D_FEAT), jnp.float32),
        pltpu.VMEM((GK, D_FEAT), jnp.float32),
        pltpu.VMEM((GK, D_FEAT), jnp.float32),
        pltpu.VMEM((GK, D_FEAT), jnp.float32),
        pltpu.VMEM((GK, D_FEAT), jnp.float32),
        pltpu.VMEM((GK, D_FEAT), jnp.float32),
        pltpu.SemaphoreType.DMA,
    ],
)
def _merge_kernel(part, hhat, dis128, dinv128, out_prev,
                  hhat_next, out_next, pa, pb, hb, d1, d2, ob, sem):
    w = _wid()

    for q in range(ROWS_W // GK):
        base = w * ROWS_W + q * GK
        sl = pl.ds(base, GK)
        pltpu.sync_copy(part.at[0, sl], pa)
        pltpu.sync_copy(part.at[1, sl], pb)
        pltpu.sync_copy(hhat.at[sl], hb)
        pltpu.sync_copy(dis128.at[sl], d1)
        pltpu.sync_copy(dinv128.at[sl], d2)
        pltpu.sync_copy(out_prev.at[sl], ob)

        def body(r, carry):
            for j in range(D_FEAT // 16):
                s = pl.ds(j * 16, 16)
                t = pa[r, s] + pb[r, s] + hb[r, s]
                ob[r, s] = ob[r, s] + (1.0 / K_HOPS) * (d1[r, s] * t)
                hb[r, s] = d2[r, s] * t
            return carry

        lax.fori_loop(0, GK, body, 0)
        pltpu.sync_copy(hb, hhat_next.at[sl])
        pltpu.sync_copy(ob, out_next.at[sl])


def kernel(x, edge_index):
    row = edge_index[0].astype(jnp.int32)
    col = edge_index[1].astype(jnp.int32)
    pad_idx = N_NODES + jnp.arange(E_PAD - N_EDGES, dtype=jnp.int32) % (
        N_PAD - N_NODES)
    row4 = jnp.concatenate([row, pad_idx]).reshape(NW, NSLAB, SGRP, HGK)
    col4 = jnp.concatenate([col, pad_idx]).reshape(NW, NSLAB, SGRP, HGK)
    col3d = col.reshape(NW, NG, GK)
    xp = jnp.pad(x, ((0, N_PAD - N_NODES), (0, 0)))

    degp = _deg_kernel(col3d)
    deg = degp[0, :, 0] + degp[1, :, 0] + 1.0
    dis = deg ** -0.5
    dinv = 1.0 / deg
    dis128 = jnp.broadcast_to(dis[:, None], (N_PAD, D_FEAT))
    dinv128 = jnp.broadcast_to(dinv[:, None], (N_PAD, D_FEAT))

    out, hhat = _init_kernel(xp, dis128)
    for _ in range(K_HOPS):
        part = _hop_kernel(hhat, row4, col4)
        hhat, out = _merge_kernel(part, hhat, dis128, dinv128, out)
    return out[:N_NODES]


# speedup vs baseline: 1.0299x; 1.0299x over previous
import functools

import jax
import jax.numpy as jnp
from jax import lax
from jax.experimental import pallas as pl
from jax.experimental.pallas import tpu as pltpu
from jax.experimental.pallas import tpu_sc as plsc

N_NODES = 10000
N_PAD = 10240
D_FEAT = 128
N_EDGES = 320000
K_HOPS = 3
ALPHA = 0.05

NC = 2
NS = 16
NW = NC * NS
E_PER_W = N_EDGES // NW
GK = 80
NG = E_PER_W // GK
E_PAD = NW * 10240
NSLAB = 2
SGRP = 64
SPAIR = SGRP // 2
ROWS_S = N_PAD // NS
ROWS_W = N_PAD // NW

_MESH = plsc.VectorSubcoreMesh(core_axis_name="c", subcore_axis_name="s")


def _wid():
    return lax.axis_index("s") * NC + lax.axis_index("c")


def _fill(ref, nrows, value):
    ncol = ref.shape[1] // 16
    v = jnp.full((16,), value, jnp.float32)

    def body(r, carry):
        for j in range(ncol):
            ref[r, pl.ds(j * 16, 16)] = v
        return carry

    lax.fori_loop(0, nrows, body, 0)


@functools.partial(
    pl.kernel,
    out_type=jax.ShapeDtypeStruct((NC, N_PAD, D_FEAT), jnp.float32),
    mesh=_MESH,
    scratch_types=[
        pltpu.VMEM((NG, GK), jnp.int32),
        pltpu.VMEM((GK, D_FEAT), jnp.float32),
        pltpu.VMEM_SHARED((N_PAD, D_FEAT), jnp.float32),
        pltpu.SemaphoreType.DMA,
    ],
)
def _deg_kernel(col3, degp, cidx, ones_v, hist, sem):
    cid = lax.axis_index("c")
    sid = lax.axis_index("s")
    w = _wid()
    pltpu.sync_copy(col3.at[w], cidx)
    _fill(ones_v, GK, 0.0)
    for q in range(ROWS_S // GK):
        pltpu.sync_copy(ones_v, hist.at[pl.ds(sid * ROWS_S + q * GK, GK)])
    plsc.subcore_barrier()
    _fill(ones_v, GK, 1.0)

    def body(g, carry):
        pltpu.sync_copy(ones_v, hist.at[cidx.at[g]], add=True)
        return carry

    lax.fori_loop(0, NG, body, 0)
    plsc.subcore_barrier()
    pltpu.sync_copy(hist.at[pl.ds(sid * ROWS_S, ROWS_S)],
                    degp.at[cid, pl.ds(sid * ROWS_S, ROWS_S)])


@functools.partial(
    pl.kernel,
    out_type=(
        jax.ShapeDtypeStruct((N_PAD, D_FEAT), jnp.float32),
        jax.ShapeDtypeStruct((N_PAD, D_FEAT), jnp.float32),
    ),
    mesh=_MESH,
    scratch_types=[
        pltpu.VMEM((GK, D_FEAT), jnp.float32),
        pltpu.VMEM((GK, D_FEAT), jnp.float32),
        pltpu.VMEM((GK, D_FEAT), jnp.float32),
        pltpu.SemaphoreType.DMA,
    ],
)
def _init_kernel(x, dis128, out, hhat, xb, db, hb, sem):
    w = _wid()

    for q in range(ROWS_W // GK):
        base = w * ROWS_W + q * GK
        pltpu.sync_copy(x.at[pl.ds(base, GK)], xb)
        pltpu.sync_copy(dis128.at[pl.ds(base, GK)], db)

        def body(r, carry):
            for j in range(D_FEAT // 16):
                s = pl.ds(j * 16, 16)
                v = xb[r, s]
                hb[r, s] = db[r, s] * v
                xb[r, s] = ALPHA * v
            return carry

        lax.fori_loop(0, GK, body, 0)
        pltpu.sync_copy(xb, out.at[pl.ds(base, GK)])
        pltpu.sync_copy(hb, hhat.at[pl.ds(base, GK)])


def _slab_phase(hhat, rs, cs, rows_a, rows_b, acc, sem_a, sem_b):
    pltpu.async_copy(hhat.at[rs.at[0]], rows_a, sem_a)

    def body(i, carry):
        g = 2 * i
        pltpu.async_copy(hhat.at[rs.at[g + 1]], rows_b, sem_b)
        pltpu.make_async_copy(hhat.at[pl.ds(0, GK)], rows_a, sem_a).wait()
        pltpu.sync_copy(rows_a, acc.at[cs.at[g]], add=True)

        g2 = jnp.minimum(g + 2, SGRP - 1)
        pltpu.async_copy(hhat.at[rs.at[g2]], rows_a, sem_a)

        pltpu.make_async_copy(hhat.at[pl.ds(0, GK)], rows_b, sem_b).wait()
        pltpu.sync_copy(rows_b, acc.at[cs.at[g + 1]], add=True)
        return carry

    lax.fori_loop(0, SPAIR, body, 0)
    pltpu.make_async_copy(hhat.at[pl.ds(0, GK)], rows_a, sem_a).wait()


@functools.partial(
    pl.kernel,
    out_type=jax.ShapeDtypeStruct((NC, N_PAD, D_FEAT), jnp.float32),
    mesh=_MESH,
    scratch_types=[
        pltpu.VMEM((SGRP, GK), jnp.int32),
        pltpu.VMEM((SGRP, GK), jnp.int32),
        pltpu.VMEM((GK, D_FEAT), jnp.float32),
        pltpu.VMEM((GK, D_FEAT), jnp.float32),
        pltpu.VMEM_SHARED((N_PAD, D_FEAT), jnp.float32),
        pltpu.SemaphoreType.DMA,
        pltpu.SemaphoreType.DMA,
    ],
)
def _hop_kernel(hhat, row4, col4, part, rs, cs, rows_a, rows_b, acc,
                sem_a, sem_b):
    cid = lax.axis_index("c")
    sid = lax.axis_index("s")
    w = _wid()
    _fill(rows_a, GK, 0.0)
    for q in range(ROWS_S // GK):
        pltpu.sync_copy(rows_a, acc.at[pl.ds(sid * ROWS_S + q * GK, GK)])
    plsc.subcore_barrier()

    for s in range(NSLAB):
        pltpu.sync_copy(row4.at[w, s], rs)
        pltpu.sync_copy(col4.at[w, s], cs)
        _slab_phase(hhat, rs, cs, rows_a, rows_b, acc, sem_a, sem_b)

    plsc.subcore_barrier()
    pltpu.sync_copy(acc.at[pl.ds(sid * ROWS_S, ROWS_S)],
                    part.at[cid, pl.ds(sid * ROWS_S, ROWS_S)])


@functools.partial(
    pl.kernel,
    out_type=(
        jax.ShapeDtypeStruct((N_PAD, D_FEAT), jnp.float32),
        jax.ShapeDtypeStruct((N_PAD, D_FEAT), jnp.float32),
    ),
    mesh=_MESH,
    scratch_types=[
        pltpu.VMEM((GK, D_FEAT), jnp.float32),
        pltpu.VMEM((GK, D_FEAT), jnp.float32),
        pltpu.VMEM((GK, D_FEAT), jnp.float32),
        pltpu.VMEM((GK, D_FEAT), jnp.float32),
        pltpu.VMEM((GK, D_FEAT), jnp.float32),
        pltpu.VMEM((GK, D_FEAT), jnp.float32),
        pltpu.SemaphoreType.DMA,
    ],
)
def _merge_kernel(part, hhat, dis128, dinv128, out_prev,
                  hhat_next, out_next, pa, pb, hb, d1, d2, ob, sem):
    w = _wid()

    for q in range(ROWS_W // GK):
        base = w * ROWS_W + q * GK
        sl = pl.ds(base, GK)
        pltpu.sync_copy(part.at[0, sl], pa)
        pltpu.sync_copy(part.at[1, sl], pb)
        pltpu.sync_copy(hhat.at[sl], hb)
        pltpu.sync_copy(dis128.at[sl], d1)
        pltpu.sync_copy(dinv128.at[sl], d2)
        pltpu.sync_copy(out_prev.at[sl], ob)

        def body(r, carry):
            for j in range(D_FEAT // 16):
                s = pl.ds(j * 16, 16)
                t = pa[r, s] + pb[r, s] + hb[r, s]
                ob[r, s] = ob[r, s] + (1.0 / K_HOPS) * (d1[r, s] * t)
                hb[r, s] = d2[r, s] * t
            return carry

        lax.fori_loop(0, GK, body, 0)
        pltpu.sync_copy(hb, hhat_next.at[sl])
        pltpu.sync_copy(ob, out_next.at[sl])


def kernel(x, edge_index):
    row = edge_index[0].astype(jnp.int32)
    col = edge_index[1].astype(jnp.int32)
    pad_idx = N_NODES + jnp.arange(E_PAD - N_EDGES, dtype=jnp.int32) % (
        N_PAD - N_NODES)
    row4 = jnp.concatenate([row, pad_idx]).reshape(NW, NSLAB, SGRP, GK)
    col4 = jnp.concatenate([col, pad_idx]).reshape(NW, NSLAB, SGRP, GK)
    col3d = col.reshape(NW, NG, GK)
    xp = jnp.pad(x, ((0, N_PAD - N_NODES), (0, 0)))

    degp = _deg_kernel(col3d)
    deg = degp[0, :, 0] + degp[1, :, 0] + 1.0
    dis = deg ** -0.5
    dinv = 1.0 / deg
    dis128 = jnp.broadcast_to(dis[:, None], (N_PAD, D_FEAT))
    dinv128 = jnp.broadcast_to(dinv[:, None], (N_PAD, D_FEAT))

    out, hhat = _init_kernel(xp, dis128)
    for _ in range(K_HOPS):
        part = _hop_kernel(hhat, row4, col4)
        hhat, out = _merge_kernel(part, hhat, dis128, dinv128, out)
    return out[:N_NODES]
